# baseline (speedup 1.0000x reference)
import jax
import jax.numpy as jnp
from jax import lax
from jax.experimental import pallas as pl
from jax.experimental.pallas import tpu as pltpu
from jax.experimental.pallas import tpu_sc as plsc

B = 16384
EM = 16
HIST = 200
NUM_SPARSE = 13
DENSE = 17

NW = 32
S_PER_W = B // NW
G = 16
IDX_PER_G = G * HIST
N_GROUPS = S_PER_W // G
S_CHUNKS = S_PER_W // 128

CHUNK = 1024
N0 = 1000001
NS = 100001
N0_FULL = N0 // CHUNK
N0_TAIL = N0 - N0_FULL * CHUNK
NS_FULL = NS // CHUNK
NS_TAIL = NS - NS_FULL * CHUNK
N0_PAD = (N0_FULL + 1) * CHUNK
NS_PAD = (NS_FULL + 1) * CHUNK

_SC_PARAMS = pltpu.CompilerParams(use_tc_tiling_on_sc=False,
                                  needs_layout_passes=False)
_SC_PARAMS_TILED = pltpu.CompilerParams(use_tc_tiling_on_sc=True,
                                        needs_layout_passes=False)


def _interleave(pbuf, obuf, lane16, nvec):
    @pl.loop(0, nvec)
    def _k(k):
        kb = lane16 + k * 256
        for e in range(16):
            plsc.store_scatter(obuf, [kb + e], pbuf[e, pl.ds(k * 16, 16)])


def _rel_table(tf_ref, tail_ref, out_ref, pb, ob, semi, semo, lane16, wid,
               nfull, ntail, rtail, tail_owner):
    kmax = (nfull - 1 + NW - 1) // NW
    nc = (nfull - 1 + NW - 1 - wid) // NW

    def issue_in(j, pi):
        c = j * NW + wid + 1

        @pl.when(c < nfull)
        def _():
            pltpu.async_copy(tf_ref.at[:, pl.ds(c * CHUNK, CHUNK)], pb[pi],
                             semi[pi])

    issue_in(0, 0)
    issue_in(1, 1)

    @pl.loop(0, (kmax + 5) // 6)
    def _jj(jj):
        for q in range(6):
            j = jj * 6 + q
            pi, po = q % 3, q % 2
            c = j * NW + wid + 1

            @pl.when(c < nfull)
            def _():
                pltpu.make_async_copy(tf_ref.at[:, pl.ds(0, CHUNK)], pb[pi],
                                      semi[pi]).wait()
                issue_in(j + 2, (q + 2) % 3)

                @pl.when(j >= 2)
                def _drain():
                    pltpu.make_async_copy(
                        ob[po], out_ref.at[pl.ds(0, CHUNK * 16)],
                        semo[po]).wait()

                _interleave(pb[pi], ob[po], lane16, CHUNK // 16)
                pltpu.async_copy(ob[po],
                                 out_ref.at[pl.ds(c * (CHUNK * 16) - 16,
                                                  CHUNK * 16)], semo[po])

    @pl.when(nc >= 1)
    def _d0():
        pltpu.make_async_copy(ob[0], out_ref.at[pl.ds(0, CHUNK * 16)],
                              semo[0]).wait()

    @pl.when(nc >= 2)
    def _d1():
        pltpu.make_async_copy(ob[1], out_ref.at[pl.ds(0, CHUNK * 16)],
                              semo[1]).wait()

    @pl.when(wid == (tail_owner + 1) % NW)
    def _chunk0():
        pltpu.sync_copy(tf_ref.at[:, pl.ds(0, CHUNK)], pb[0])
        _interleave(pb[0], ob[0], lane16, CHUNK // 16)
        pltpu.sync_copy(ob[0].at[pl.ds(16, CHUNK * 16 - 16)],
                        out_ref.at[pl.ds(0, CHUNK * 16 - 16)])

    @pl.when(wid == tail_owner)
    def _tail():
        pltpu.sync_copy(tail_ref, pb[1].at[:, pl.ds(0, rtail)])
        _interleave(pb[1], ob[1], lane16, (ntail + 15) // 16)
        pltpu.sync_copy(ob[1].at[pl.ds(0, ntail * 16)],
                        out_ref.at[pl.ds(nfull * (CHUNK * 16) - 16,
                                         ntail * 16)])


def _rel0_fn(tf_ref, tail_ref, out_ref, pb0, pb1, pb2, ob0, ob1, si0, si1,
             si2, so0, so1):
    wid = lax.axis_index("subcore") * 2 + lax.axis_index("core")
    lane16 = lax.iota(jnp.int32, 16) * 16
    _rel_table(tf_ref, tail_ref, out_ref, (pb0, pb1, pb2), (ob0, ob1),
               (si0, si1, si2), (so0, so1), lane16, wid, N0_FULL, N0_TAIL,
               640, 7)


def _rels_fn(t1, t2, t3, t4, t5, t6, t7, t8, t9, t10, t11, t12,
             x1, x2, x3, x4, x5, x6, x7, x8, x9, x10, x11, x12,
             o1, o2, o3, o4, o5, o6, o7, o8, o9, o10, o11, o12,
             pb0, pb1, pb2, ob0, ob1, si0, si1, si2, so0, so1):
    wid = lax.axis_index("subcore") * 2 + lax.axis_index("core")
    lane16 = lax.iota(jnp.int32, 16) * 16
    ins = (t1, t2, t3, t4, t5, t6, t7, t8, t9, t10, t11, t12)
    tails = (x1, x2, x3, x4, x5, x6, x7, x8, x9, x10, x11, x12)
    outs = (o1, o2, o3, o4, o5, o6, o7, o8, o9, o10, o11, o12)
    pb, ob = (pb0, pb1, pb2), (ob0, ob1)
    semi, semo = (si0, si1, si2), (so0, so1)

    items = [(ins[i], outs[i], j) for i in range(len(ins)) for j in range(3)]

    def issue_in(n):
        tf_ref, _, j = items[n]
        c = j * NW + wid + 1
        pltpu.async_copy(tf_ref.at[:, pl.ds(c * CHUNK, CHUNK)], pb[n % 3],
                         semi[n % 3])

    issue_in(0)
    issue_in(1)
    for n, (tf_ref, out_ref, j) in enumerate(items):
        pi, po = n % 3, n % 2
        c = j * NW + wid + 1
        pltpu.make_async_copy(tf_ref.at[:, pl.ds(0, CHUNK)], pb[pi],
                              semi[pi]).wait()
        if n + 2 < len(items):
            issue_in(n + 2)
        if n >= 2:
            pltpu.make_async_copy(ob[po], out_ref.at[pl.ds(0, CHUNK * 16)],
                                  semo[po]).wait()
        _interleave(pb[pi], ob[po], lane16, CHUNK // 16)
        pltpu.async_copy(ob[po], out_ref.at[pl.ds(c * (CHUNK * 16) - 16,
                                                  CHUNK * 16)], semo[po])

    for po in (0, 1):
        pltpu.make_async_copy(ob[po], outs[0].at[pl.ds(0, CHUNK * 16)],
                              semo[po]).wait()

    for i in range(len(ins)):
        @pl.when(wid == 3 + i)
        def _chunk0(tf_ref=ins[i], out_ref=outs[i]):
            pltpu.sync_copy(tf_ref.at[:, pl.ds(0, CHUNK)], pb[0])
            _interleave(pb[0], ob[0], lane16, CHUNK // 16)
            pltpu.sync_copy(ob[0].at[pl.ds(16, CHUNK * 16 - 16)],
                            out_ref.at[pl.ds(0, CHUNK * 16 - 16)])

        @pl.when(wid == 2 + i)
        def _tail(tail_ref=tails[i], out_ref=outs[i]):
            pltpu.sync_copy(tail_ref, pb[1].at[:, pl.ds(0, 768)])
            _interleave(pb[1], ob[1], lane16, (NS_TAIL + 15) // 16)
            pltpu.sync_copy(ob[1].at[pl.ds(0, NS_TAIL * 16)],
                            out_ref.at[pl.ds(NS_FULL * (CHUNK * 16) - 16,
                                             NS_TAIL * 16)])


def _hist_fn(uch_ref, t0_ref, out_ref, hidx0, hidx1, rows0, rows1, fh_v,
             semg0, semg1, semi0, semi1):
    wid = lax.axis_index("subcore") * 2 + lax.axis_index("core")
    base = wid * S_PER_W
    idx0 = wid * (S_PER_W * HIST)
    zero = jnp.zeros((16,), jnp.float32)
    hidx = (hidx0, hidx1)
    rows = (rows0, rows1)
    semg = (semg0, semg1)
    semi = (semi0, semi1)

    def idx_copy(g, p, sync):
        src = uch_ref.at[pl.ds(idx0 + g * IDX_PER_G, IDX_PER_G)]
        if sync:
            pltpu.sync_copy(src, hidx[p])
        else:
            pltpu.async_copy(src, hidx[p], semi[p])

    def fire(g, p):
        for j in range(IDX_PER_G // 128):
            pltpu.async_copy(t0_ref.at[hidx[p].at[pl.ds(j * 128, 128)]],
                             rows[p].at[pl.ds(j * 128, 128)], semg[p])

    def reduce(g, p):
        @pl.loop(0, G)
        def _sample(s):
            def body(j, accs):
                a0, a1 = accs
                o = s * HIST + j * 8
                for t in range(4):
                    a0 = a0 + rows[p][o + 2 * t]
                    a1 = a1 + rows[p][o + 2 * t + 1]
                return (a0, a1)

            a0, a1 = lax.fori_loop(0, HIST // 8, body, (zero, zero))
            fh_v[g * G + s] = a0 + a1

    idx_copy(0, 0, sync=True)
    fire(0, 0)
    idx_copy(1, 1, sync=False)

    @pl.loop(0, N_GROUPS // 2)
    def _g2(k):
        for p in range(2):
            g = k * 2 + p

            @pl.when(g + 1 < N_GROUPS)
            def _():
                pltpu.make_async_copy(
                    uch_ref.at[pl.ds(0, IDX_PER_G)], hidx[1 - p],
                    semi[1 - p]).wait()
                fire(g + 1, 1 - p)

            pltpu.make_async_copy(t0_ref.at[pl.ds(0, IDX_PER_G)], rows[p],
                                  semg[p]).wait()

            @pl.when(g + 2 < N_GROUPS)
            def _():
                idx_copy(g + 2, p, sync=False)

            reduce(g, p)

    pltpu.sync_copy(fh_v, out_ref.at[pl.ds(base, S_PER_W)])


def _sparse_fn(sp_ref, t0, t1, t2, t3, t4, t5, t6, t7, t8, t9, t10, t11, t12,
               out_ref, sidx0, sidx1, srows0, srows1, scat0, scat1,
               semg0, semg1, sems0, sems1):
    tables = (t0, t1, t2, t3, t4, t5, t6, t7, t8, t9, t10, t11, t12)
    wid = lax.axis_index("subcore") * 2 + lax.axis_index("core")
    base = wid * S_PER_W
    lane = lax.iota(jnp.int32, 16)
    sidx = (sidx0, sidx1)
    srows = (srows0, srows1)
    scat = (scat0, scat1)
    semg = (semg0, semg1)
    sems = (sems0, sems1)

    def fire_gathers(i, p):
        pltpu.sync_copy(sp_ref.at[pl.ds(i * B + base, S_PER_W)], sidx[p])
        for r in range(S_CHUNKS):
            pltpu.async_copy(tables[i].at[sidx[p].at[pl.ds(r * 128, 128)]],
                             srows[p].at[pl.ds(r * 128, 128)], semg[p])

    fire_gathers(0, 0)
    for i in range(NUM_SPARSE):
        p = i % 2
        pltpu.make_async_copy(tables[i].at[pl.ds(0, S_PER_W)], srows[p],
                              semg[p]).wait()
        if i >= 1:
            for r in range(S_CHUNKS):
                pltpu.make_async_copy(srows[1 - p].at[pl.ds(r * 128, 128)],
                                      out_ref.at[scat[1 - p].at[r]],
                                      sems[1 - p]).wait()
        if i + 1 < NUM_SPARSE:
            fire_gathers(i + 1, 1 - p)

        @pl.loop(0, S_CHUNKS)
        def _fr(r):
            @pl.loop(0, 128, step=16)
            def _fc(c):
                k = base + r * 128 + c + lane
                scat[p][r, pl.ds(c, 16)] = k * NUM_SPARSE + i

        for r in range(S_CHUNKS):
            pltpu.async_copy(srows[p].at[pl.ds(r * 128, 128)],
                             out_ref.at[scat[p].at[r]], sems[p])

    for r in range(S_CHUNKS):
        pltpu.make_async_copy(srows[0].at[pl.ds(r * 128, 128)],
                              out_ref.at[scat[0].at[r]], sems[0]).wait()


def _mlp_fn(fs_ref, h_ref, d_ref, w1s_ref, w1h_ref, w1d_ref, b1_ref, w2_ref,
            b2_ref, w3_ref, b3_ref, o_ref):
    h = jnp.dot(fs_ref[...], w1s_ref[...], preferred_element_type=jnp.float32)
    h = h + jnp.dot(h_ref[...], w1h_ref[...],
                    preferred_element_type=jnp.float32)
    h = h + jnp.dot(d_ref[...], w1d_ref[...],
                    preferred_element_type=jnp.float32)
    h = jnp.maximum(h + b1_ref[...], 0.0)
    h2 = jnp.dot(h, w2_ref[...], preferred_element_type=jnp.float32)
    h2 = jnp.maximum(h2 + b2_ref[...], 0.0)
    o_ref[...] = (jnp.dot(h2, w3_ref[...], preferred_element_type=jnp.float32)
                  + b3_ref[...])


_mesh = plsc.VectorSubcoreMesh(core_axis_name="core",
                               subcore_axis_name="subcore")

_REL_SCRATCH = [
    pltpu.VMEM((16, CHUNK), jnp.float32),
    pltpu.VMEM((16, CHUNK), jnp.float32),
    pltpu.VMEM((16, CHUNK), jnp.float32),
    pltpu.VMEM((CHUNK * 16,), jnp.float32),
    pltpu.VMEM((CHUNK * 16,), jnp.float32),
    pltpu.SemaphoreType.DMA,
    pltpu.SemaphoreType.DMA,
    pltpu.SemaphoreType.DMA,
    pltpu.SemaphoreType.DMA,
    pltpu.SemaphoreType.DMA,
]

_rel0 = pl.kernel(
    _rel0_fn,
    out_type=jax.ShapeDtypeStruct((N0_PAD * 16,), jnp.float32),
    mesh=_mesh,
    scratch_types=list(_REL_SCRATCH),
    compiler_params=_SC_PARAMS_TILED,
)

_rels = pl.kernel(
    _rels_fn,
    out_type=[jax.ShapeDtypeStruct((NS_PAD * 16,), jnp.float32)] * 12,
    mesh=_mesh,
    scratch_types=list(_REL_SCRATCH),
    compiler_params=_SC_PARAMS_TILED,
)

_hist = pl.kernel(
    _hist_fn,
    out_type=jax.ShapeDtypeStruct((B, EM), jnp.float32),
    mesh=_mesh,
    scratch_types=[
        pltpu.VMEM((IDX_PER_G,), jnp.int32),
        pltpu.VMEM((IDX_PER_G,), jnp.int32),
        pltpu.VMEM((IDX_PER_G, EM), jnp.float32),
        pltpu.VMEM((IDX_PER_G, EM), jnp.float32),
        pltpu.VMEM((S_PER_W, EM), jnp.float32),
        pltpu.SemaphoreType.DMA,
        pltpu.SemaphoreType.DMA,
        pltpu.SemaphoreType.DMA,
        pltpu.SemaphoreType.DMA,
    ],
    compiler_params=_SC_PARAMS,
)

_sparse = pl.kernel(
    _sparse_fn,
    out_type=jax.ShapeDtypeStruct((NUM_SPARSE * B, EM), jnp.float32),
    mesh=_mesh,
    scratch_types=[
        pltpu.VMEM((S_PER_W,), jnp.int32),
        pltpu.VMEM((S_PER_W,), jnp.int32),
        pltpu.VMEM((S_PER_W, EM), jnp.float32),
        pltpu.VMEM((S_PER_W, EM), jnp.float32),
        pltpu.VMEM((S_CHUNKS, 128), jnp.int32),
        pltpu.VMEM((S_CHUNKS, 128), jnp.int32),
        pltpu.SemaphoreType.DMA,
        pltpu.SemaphoreType.DMA,
        pltpu.SemaphoreType.DMA,
        pltpu.SemaphoreType.DMA,
    ],
    compiler_params=_SC_PARAMS,
)


def kernel(sparse_features, dense_features, user_click_history, tables,
           fc1_w, fc1_b, fc2_w, fc2_b, fc3_w, fc3_b):
    uch1 = user_click_history.reshape(-1)
    sp1 = sparse_features.T.reshape(-1)

    t0t = tables[0].T
    tail0 = jnp.pad(t0t[:, N0_FULL * CHUNK:], ((0, 0), (0, 640 - N0_TAIL)))
    t0r = _rel0(t0t, tail0).reshape(N0_PAD, EM)
    stv = [t.T for t in tables[1:]]
    stails = [jnp.pad(t[:, NS_FULL * CHUNK:], ((0, 0), (0, 768 - NS_TAIL)))
              for t in stv]
    smalls = _rels(*stv, *stails)
    smalls = [s.reshape(NS_PAD, EM) for s in smalls]

    hist = _hist(uch1, t0r)
    featS = _sparse(sp1, t0r, *smalls).reshape(B, NUM_SPARSE * EM)

    w1s = fc1_w[:, :NUM_SPARSE * EM].T
    w1h = fc1_w[:, NUM_SPARSE * EM:(NUM_SPARSE + 1) * EM].T
    w1d = fc1_w[:, (NUM_SPARSE + 1) * EM:].T
    w2t = fc2_w.T
    w3t = fc3_w.T
    b1r = fc1_b.reshape(1, -1)
    b2r = fc2_b.reshape(1, -1)
    b3r = fc3_b.reshape(1, -1)

    BLK = 2048
    out = pl.pallas_call(
        _mlp_fn,
        grid=(B // BLK,),
        in_specs=[
            pl.BlockSpec((BLK, NUM_SPARSE * EM), lambda i: (i, 0)),
            pl.BlockSpec((BLK, EM), lambda i: (i, 0)),
            pl.BlockSpec((BLK, DENSE), lambda i: (i, 0)),
            pl.BlockSpec(w1s.shape, lambda i: (0, 0)),
            pl.BlockSpec(w1h.shape, lambda i: (0, 0)),
            pl.BlockSpec(w1d.shape, lambda i: (0, 0)),
            pl.BlockSpec(b1r.shape, lambda i: (0, 0)),
            pl.BlockSpec(w2t.shape, lambda i: (0, 0)),
            pl.BlockSpec(b2r.shape, lambda i: (0, 0)),
            pl.BlockSpec(w3t.shape, lambda i: (0, 0)),
            pl.BlockSpec(b3r.shape, lambda i: (0, 0)),
        ],
        out_specs=pl.BlockSpec((BLK, 2), lambda i: (i, 0)),
        out_shape=jax.ShapeDtypeStruct((B, 2), jnp.float32),
    )(featS, hist, dense_features, w1s, w1h, w1d, b1r, w2t, b2r, w3t, b3r)
    return out


# speedup vs baseline: 1.0064x; 1.0064x over previous
import jax
import jax.numpy as jnp
from jax import lax
from jax.experimental import pallas as pl
from jax.experimental.pallas import tpu as pltpu
from jax.experimental.pallas import tpu_sc as plsc

B = 16384
EM = 16
HIST = 200
NUM_SPARSE = 13
DENSE = 17

NW = 32
S_PER_W = B // NW
G = 16
IDX_PER_G = G * HIST
N_GROUPS = S_PER_W // G
S_CHUNKS = S_PER_W // 128

CHUNK = 1024
N0 = 1000001
NS = 100001
N0_FULL = N0 // CHUNK
N0_TAIL = N0 - N0_FULL * CHUNK
NS_FULL = NS // CHUNK
NS_TAIL = NS - NS_FULL * CHUNK
N0_PAD = (N0_FULL + 1) * CHUNK
NS_PAD = (NS_FULL + 1) * CHUNK

_SC_PARAMS = pltpu.CompilerParams(use_tc_tiling_on_sc=False,
                                  needs_layout_passes=False)
_SC_PARAMS_TILED = pltpu.CompilerParams(use_tc_tiling_on_sc=True,
                                        needs_layout_passes=False)


def _interleave(pbuf, obuf, lane16, nvec):
    @pl.loop(0, nvec)
    def _k(k):
        kb = lane16 + k * 256
        for e in range(16):
            plsc.store_scatter(obuf, [kb + e], pbuf[e, pl.ds(k * 16, 16)])


def _rel_table(tf_ref, tail_ref, out_ref, pb, ob, semi, semo, lane16, wid,
               nfull, ntail, rtail, tail_owner):
    kmax = (nfull - 1 + NW - 1) // NW
    nc = (nfull - 1 + NW - 1 - wid) // NW

    def issue_in(j, p):
        c = j * NW + wid + 1

        @pl.when(c < nfull)
        def _():
            pltpu.async_copy(tf_ref.at[:, pl.ds(c * CHUNK, CHUNK)], pb[p],
                             semi[p])

    issue_in(0, 0)

    @pl.loop(0, (kmax + 1) // 2)
    def _jj(jj):
        for p in range(2):
            j = jj * 2 + p
            c = j * NW + wid + 1

            @pl.when(c < nfull)
            def _():
                pltpu.make_async_copy(tf_ref.at[:, pl.ds(0, CHUNK)], pb[p],
                                      semi[p]).wait()
                issue_in(j + 1, 1 - p)

                @pl.when(jj >= 1)
                def _drain():
                    pltpu.make_async_copy(
                        ob[p], out_ref.at[pl.ds(0, CHUNK * 16)],
                        semo[p]).wait()

                _interleave(pb[p], ob[p], lane16, CHUNK // 16)
                pltpu.async_copy(ob[p],
                                 out_ref.at[pl.ds(c * (CHUNK * 16) - 16,
                                                  CHUNK * 16)], semo[p])

    @pl.when(nc >= 1)
    def _d0():
        pltpu.make_async_copy(ob[0], out_ref.at[pl.ds(0, CHUNK * 16)],
                              semo[0]).wait()

    @pl.when(nc >= 2)
    def _d1():
        pltpu.make_async_copy(ob[1], out_ref.at[pl.ds(0, CHUNK * 16)],
                              semo[1]).wait()

    @pl.when(wid == (tail_owner + 1) % NW)
    def _chunk0():
        pltpu.sync_copy(tf_ref.at[:, pl.ds(0, CHUNK)], pb[0])
        _interleave(pb[0], ob[0], lane16, CHUNK // 16)
        pltpu.sync_copy(ob[0].at[pl.ds(16, CHUNK * 16 - 16)],
                        out_ref.at[pl.ds(0, CHUNK * 16 - 16)])

    @pl.when(wid == tail_owner)
    def _tail():
        pltpu.sync_copy(tail_ref, pb[1].at[:, pl.ds(0, rtail)])
        _interleave(pb[1], ob[1], lane16, (ntail + 15) // 16)
        pltpu.sync_copy(ob[1].at[pl.ds(0, ntail * 16)],
                        out_ref.at[pl.ds(nfull * (CHUNK * 16) - 16,
                                         ntail * 16)])


def _rel0_fn(tf_ref, tail_ref, out_ref, pb0, pb1, ob0, ob1, si0, si1, so0, so1):
    wid = lax.axis_index("subcore") * 2 + lax.axis_index("core")
    lane16 = lax.iota(jnp.int32, 16) * 16
    _rel_table(tf_ref, tail_ref, out_ref, (pb0, pb1), (ob0, ob1), (si0, si1),
               (so0, so1), lane16, wid, N0_FULL, N0_TAIL, 640, 7)


def _rels_fn(t1, t2, t3, t4, t5, t6, t7, t8, t9, t10, t11, t12,
             x1, x2, x3, x4, x5, x6, x7, x8, x9, x10, x11, x12,
             o1, o2, o3, o4, o5, o6, o7, o8, o9, o10, o11, o12,
             pb0, pb1, ob0, ob1, si0, si1, so0, so1):
    wid = lax.axis_index("subcore") * 2 + lax.axis_index("core")
    lane16 = lax.iota(jnp.int32, 16) * 16
    ins = (t1, t2, t3, t4, t5, t6, t7, t8, t9, t10, t11, t12)
    tails = (x1, x2, x3, x4, x5, x6, x7, x8, x9, x10, x11, x12)
    outs = (o1, o2, o3, o4, o5, o6, o7, o8, o9, o10, o11, o12)
    pb, ob = (pb0, pb1), (ob0, ob1)
    semi, semo = (si0, si1), (so0, so1)

    items = [(ins[i], outs[i], j) for i in range(len(ins)) for j in range(3)]

    def issue_in(n, p):
        tf_ref, _, j = items[n]
        c = j * NW + wid + 1
        pltpu.async_copy(tf_ref.at[:, pl.ds(c * CHUNK, CHUNK)], pb[p],
                         semi[p])

    issue_in(0, 0)
    for n, (tf_ref, out_ref, j) in enumerate(items):
        p = n % 2
        c = j * NW + wid + 1
        pltpu.make_async_copy(tf_ref.at[:, pl.ds(0, CHUNK)], pb[p],
                              semi[p]).wait()
        if n + 1 < len(items):
            issue_in(n + 1, 1 - p)
        if n >= 2:
            pltpu.make_async_copy(ob[p], out_ref.at[pl.ds(0, CHUNK * 16)],
                                  semo[p]).wait()
        _interleave(pb[p], ob[p], lane16, CHUNK // 16)
        pltpu.async_copy(ob[p], out_ref.at[pl.ds(c * (CHUNK * 16) - 16,
                                                 CHUNK * 16)], semo[p])

    for p in (0, 1):
        pltpu.make_async_copy(ob[p], outs[0].at[pl.ds(0, CHUNK * 16)],
                              semo[p]).wait()

    for i in range(len(ins)):
        @pl.when(wid == 3 + i)
        def _chunk0(tf_ref=ins[i], out_ref=outs[i]):
            pltpu.sync_copy(tf_ref.at[:, pl.ds(0, CHUNK)], pb[0])
            _interleave(pb[0], ob[0], lane16, CHUNK // 16)
            pltpu.sync_copy(ob[0].at[pl.ds(16, CHUNK * 16 - 16)],
                            out_ref.at[pl.ds(0, CHUNK * 16 - 16)])

        @pl.when(wid == 2 + i)
        def _tail(tail_ref=tails[i], out_ref=outs[i]):
            pltpu.sync_copy(tail_ref, pb[1].at[:, pl.ds(0, 768)])
            _interleave(pb[1], ob[1], lane16, (NS_TAIL + 15) // 16)
            pltpu.sync_copy(ob[1].at[pl.ds(0, NS_TAIL * 16)],
                            out_ref.at[pl.ds(NS_FULL * (CHUNK * 16) - 16,
                                             NS_TAIL * 16)])


def _hist_fn(uch_ref, t0_ref, out_ref, hidx0, hidx1, rows0, rows1, fh_v,
             semg0, semg1, semi0, semi1):
    wid = lax.axis_index("subcore") * 2 + lax.axis_index("core")
    base = wid * S_PER_W
    idx0 = wid * (S_PER_W * HIST)
    zero = jnp.zeros((16,), jnp.float32)
    hidx = (hidx0, hidx1)
    rows = (rows0, rows1)
    semg = (semg0, semg1)
    semi = (semi0, semi1)

    def idx_copy(g, p, sync):
        src = uch_ref.at[pl.ds(idx0 + g * IDX_PER_G, IDX_PER_G)]
        if sync:
            pltpu.sync_copy(src, hidx[p])
        else:
            pltpu.async_copy(src, hidx[p], semi[p])

    def fire(g, p):
        for j in range(IDX_PER_G // 128):
            pltpu.async_copy(t0_ref.at[hidx[p].at[pl.ds(j * 128, 128)]],
                             rows[p].at[pl.ds(j * 128, 128)], semg[p])

    def reduce(g, p):
        @pl.loop(0, G)
        def _sample(s):
            def body(j, accs):
                a0, a1 = accs
                o = s * HIST + j * 8
                for t in range(4):
                    a0 = a0 + rows[p][o + 2 * t]
                    a1 = a1 + rows[p][o + 2 * t + 1]
                return (a0, a1)

            a0, a1 = lax.fori_loop(0, HIST // 8, body, (zero, zero))
            fh_v[g * G + s] = a0 + a1

    idx_copy(0, 0, sync=True)
    fire(0, 0)
    idx_copy(1, 1, sync=False)

    @pl.loop(0, N_GROUPS // 2)
    def _g2(k):
        for p in range(2):
            g = k * 2 + p

            @pl.when(g + 1 < N_GROUPS)
            def _():
                pltpu.make_async_copy(
                    uch_ref.at[pl.ds(0, IDX_PER_G)], hidx[1 - p],
                    semi[1 - p]).wait()
                fire(g + 1, 1 - p)

            pltpu.make_async_copy(t0_ref.at[pl.ds(0, IDX_PER_G)], rows[p],
                                  semg[p]).wait()

            @pl.when(g + 2 < N_GROUPS)
            def _():
                idx_copy(g + 2, p, sync=False)

            reduce(g, p)

    pltpu.sync_copy(fh_v, out_ref.at[pl.ds(base, S_PER_W)])


def _sparse_fn(sp_ref, t0, t1, t2, t3, t4, t5, t6, t7, t8, t9, t10, t11, t12,
               out_ref, sidx0, sidx1, srows0, srows1, scat0, scat1,
               semg0, semg1, sems0, sems1):
    tables = (t0, t1, t2, t3, t4, t5, t6, t7, t8, t9, t10, t11, t12)
    wid = lax.axis_index("subcore") * 2 + lax.axis_index("core")
    base = wid * S_PER_W
    lane = lax.iota(jnp.int32, 16)
    sidx = (sidx0, sidx1)
    srows = (srows0, srows1)
    scat = (scat0, scat1)
    semg = (semg0, semg1)
    sems = (sems0, sems1)

    def fire_gathers(i, p):
        pltpu.sync_copy(sp_ref.at[pl.ds(i * B + base, S_PER_W)], sidx[p])
        for r in range(S_CHUNKS):
            pltpu.async_copy(tables[i].at[sidx[p].at[pl.ds(r * 128, 128)]],
                             srows[p].at[pl.ds(r * 128, 128)], semg[p])

    fire_gathers(0, 0)
    for i in range(NUM_SPARSE):
        p = i % 2
        pltpu.make_async_copy(tables[i].at[pl.ds(0, S_PER_W)], srows[p],
                              semg[p]).wait()
        if i >= 1:
            for r in range(S_CHUNKS):
                pltpu.make_async_copy(srows[1 - p].at[pl.ds(r * 128, 128)],
                                      out_ref.at[scat[1 - p].at[r]],
                                      sems[1 - p]).wait()
        if i + 1 < NUM_SPARSE:
            fire_gathers(i + 1, 1 - p)

        @pl.loop(0, S_CHUNKS)
        def _fr(r):
            @pl.loop(0, 128, step=16)
            def _fc(c):
                k = base + r * 128 + c + lane
                scat[p][r, pl.ds(c, 16)] = k * NUM_SPARSE + i

        for r in range(S_CHUNKS):
            pltpu.async_copy(srows[p].at[pl.ds(r * 128, 128)],
                             out_ref.at[scat[p].at[r]], sems[p])

    for r in range(S_CHUNKS):
        pltpu.make_async_copy(srows[0].at[pl.ds(r * 128, 128)],
                              out_ref.at[scat[0].at[r]], sems[0]).wait()


def _mlp_fn(fs_ref, h_ref, d_ref, w1s_ref, w1h_ref, w1d_ref, b1_ref, w2_ref,
            b2_ref, w3_ref, b3_ref, o_ref):
    h = jnp.dot(fs_ref[...], w1s_ref[...], preferred_element_type=jnp.float32)
    h = h + jnp.dot(h_ref[...], w1h_ref[...],
                    preferred_element_type=jnp.float32)
    h = h + jnp.dot(d_ref[...], w1d_ref[...],
                    preferred_element_type=jnp.float32)
    h = jnp.maximum(h + b1_ref[...], 0.0)
    h2 = jnp.dot(h, w2_ref[...], preferred_element_type=jnp.float32)
    h2 = jnp.maximum(h2 + b2_ref[...], 0.0)
    o_ref[...] = (jnp.dot(h2, w3_ref[...], preferred_element_type=jnp.float32)
                  + b3_ref[...])


_mesh = plsc.VectorSubcoreMesh(core_axis_name="core",
                               subcore_axis_name="subcore")

_REL_SCRATCH = [
    pltpu.VMEM((16, CHUNK), jnp.float32),
    pltpu.VMEM((16, CHUNK), jnp.float32),
    pltpu.VMEM((CHUNK * 16,), jnp.float32),
    pltpu.VMEM((CHUNK * 16,), jnp.float32),
    pltpu.SemaphoreType.DMA,
    pltpu.SemaphoreType.DMA,
    pltpu.SemaphoreType.DMA,
    pltpu.SemaphoreType.DMA,
]

_rel0 = pl.kernel(
    _rel0_fn,
    out_type=jax.ShapeDtypeStruct((N0_PAD * 16,), jnp.float32),
    mesh=_mesh,
    scratch_types=list(_REL_SCRATCH),
    compiler_params=_SC_PARAMS_TILED,
)

_rels = pl.kernel(
    _rels_fn,
    out_type=[jax.ShapeDtypeStruct((NS_PAD * 16,), jnp.float32)] * 12,
    mesh=_mesh,
    scratch_types=list(_REL_SCRATCH),
    compiler_params=_SC_PARAMS_TILED,
)

_hist = pl.kernel(
    _hist_fn,
    out_type=jax.ShapeDtypeStruct((B, EM), jnp.float32),
    mesh=_mesh,
    scratch_types=[
        pltpu.VMEM((IDX_PER_G,), jnp.int32),
        pltpu.VMEM((IDX_PER_G,), jnp.int32),
        pltpu.VMEM((IDX_PER_G, EM), jnp.float32),
        pltpu.VMEM((IDX_PER_G, EM), jnp.float32),
        pltpu.VMEM((S_PER_W, EM), jnp.float32),
        pltpu.SemaphoreType.DMA,
        pltpu.SemaphoreType.DMA,
        pltpu.SemaphoreType.DMA,
        pltpu.SemaphoreType.DMA,
    ],
    compiler_params=_SC_PARAMS,
)

_sparse = pl.kernel(
    _sparse_fn,
    out_type=jax.ShapeDtypeStruct((NUM_SPARSE * B, EM), jnp.float32),
    mesh=_mesh,
    scratch_types=[
        pltpu.VMEM((S_PER_W,), jnp.int32),
        pltpu.VMEM((S_PER_W,), jnp.int32),
        pltpu.VMEM((S_PER_W, EM), jnp.float32),
        pltpu.VMEM((S_PER_W, EM), jnp.float32),
        pltpu.VMEM((S_CHUNKS, 128), jnp.int32),
        pltpu.VMEM((S_CHUNKS, 128), jnp.int32),
        pltpu.SemaphoreType.DMA,
        pltpu.SemaphoreType.DMA,
        pltpu.SemaphoreType.DMA,
        pltpu.SemaphoreType.DMA,
    ],
    compiler_params=_SC_PARAMS,
)


def kernel(sparse_features, dense_features, user_click_history, tables,
           fc1_w, fc1_b, fc2_w, fc2_b, fc3_w, fc3_b):
    uch1 = user_click_history.reshape(-1)
    sp1 = sparse_features.T.reshape(-1)

    t0t = tables[0].T
    tail0 = jnp.pad(t0t[:, N0_FULL * CHUNK:], ((0, 0), (0, 640 - N0_TAIL)))
    t0r = _rel0(t0t, tail0).reshape(N0_PAD, EM)
    stv = [t.T for t in tables[1:]]
    stails = [jnp.pad(t[:, NS_FULL * CHUNK:], ((0, 0), (0, 768 - NS_TAIL)))
              for t in stv]
    smalls = _rels(*stv, *stails)
    smalls = [s.reshape(NS_PAD, EM) for s in smalls]

    hist = _hist(uch1, t0r)
    featS = _sparse(sp1, t0r, *smalls).reshape(B, NUM_SPARSE * EM)

    w1s = fc1_w[:, :NUM_SPARSE * EM].T
    w1h = fc1_w[:, NUM_SPARSE * EM:(NUM_SPARSE + 1) * EM].T
    w1d = fc1_w[:, (NUM_SPARSE + 1) * EM:].T
    w2t = fc2_w.T
    w3t = fc3_w.T
    b1r = fc1_b.reshape(1, -1)
    b2r = fc2_b.reshape(1, -1)
    b3r = fc3_b.reshape(1, -1)

    BLK = 2048
    out = pl.pallas_call(
        _mlp_fn,
        grid=(B // BLK,),
        in_specs=[
            pl.BlockSpec((BLK, NUM_SPARSE * EM), lambda i: (i, 0)),
            pl.BlockSpec((BLK, EM), lambda i: (i, 0)),
            pl.BlockSpec((BLK, DENSE), lambda i: (i, 0)),
            pl.BlockSpec(w1s.shape, lambda i: (0, 0)),
            pl.BlockSpec(w1h.shape, lambda i: (0, 0)),
            pl.BlockSpec(w1d.shape, lambda i: (0, 0)),
            pl.BlockSpec(b1r.shape, lambda i: (0, 0)),
            pl.BlockSpec(w2t.shape, lambda i: (0, 0)),
            pl.BlockSpec(b2r.shape, lambda i: (0, 0)),
            pl.BlockSpec(w3t.shape, lambda i: (0, 0)),
            pl.BlockSpec(b3r.shape, lambda i: (0, 0)),
        ],
        out_specs=pl.BlockSpec((BLK, 2), lambda i: (i, 0)),
        out_shape=jax.ShapeDtypeStruct((B, 2), jnp.float32),
    )(featS, hist, dense_features, w1s, w1h, w1d, b1r, w2t, b2r, w3t, b3r)
    return out


# speedup vs baseline: 1.0222x; 1.0157x over previous
import jax
import jax.numpy as jnp
from jax import lax
from jax.experimental import pallas as pl
from jax.experimental.pallas import tpu as pltpu
from jax.experimental.pallas import tpu_sc as plsc

B = 16384
EM = 16
HIST = 200
NUM_SPARSE = 13
DENSE = 17

NW = 32
S_PER_W = B // NW
G = 16
IDX_PER_G = G * HIST
N_GROUPS = S_PER_W // G
S_CHUNKS = S_PER_W // 128

CHUNK = 1024
N0 = 1000001
NS = 100001
N0_FULL = N0 // CHUNK
N0_TAIL = N0 - N0_FULL * CHUNK
NS_FULL = NS // CHUNK
NS_TAIL = NS - NS_FULL * CHUNK
N0_PAD = (N0_FULL + 1) * CHUNK
NS_PAD = (NS_FULL + 1) * CHUNK

_SC_PARAMS = pltpu.CompilerParams(use_tc_tiling_on_sc=False,
                                  needs_layout_passes=False)
_SC_PARAMS_TILED = pltpu.CompilerParams(use_tc_tiling_on_sc=True,
                                        needs_layout_passes=False)


def _interleave(pbuf, obuf, lane16, nvec):
    @pl.loop(0, nvec)
    def _k(k):
        kb = lane16 + k * 256
        for e in range(16):
            plsc.store_scatter(obuf, [kb + e], pbuf[e, pl.ds(k * 16, 16)])


def _rel_table(tf_ref, tail_ref, out_ref, pb, ob, semi, semo, lane16, wid,
               nfull, ntail, rtail, tail_owner):
    kmax = (nfull - 1 + NW - 1) // NW
    nc = (nfull - 1 + NW - 1 - wid) // NW

    def issue_in(j, p):
        c = j * NW + wid + 1

        @pl.when(c < nfull)
        def _():
            pltpu.async_copy(tf_ref.at[:, pl.ds(c * CHUNK, CHUNK)], pb[p],
                             semi[p])

    issue_in(0, 0)

    @pl.loop(0, (kmax + 1) // 2)
    def _jj(jj):
        for p in range(2):
            j = jj * 2 + p
            c = j * NW + wid + 1

            @pl.when(c < nfull)
            def _():
                pltpu.make_async_copy(tf_ref.at[:, pl.ds(0, CHUNK)], pb[p],
                                      semi[p]).wait()
                issue_in(j + 1, 1 - p)

                @pl.when(jj >= 1)
                def _drain():
                    pltpu.make_async_copy(
                        ob[p], out_ref.at[pl.ds(0, CHUNK * 16)],
                        semo[p]).wait()

                _interleave(pb[p], ob[p], lane16, CHUNK // 16)
                pltpu.async_copy(ob[p],
                                 out_ref.at[pl.ds(c * (CHUNK * 16) - 16,
                                                  CHUNK * 16)], semo[p])

    @pl.when(nc >= 1)
    def _d0():
        pltpu.make_async_copy(ob[0], out_ref.at[pl.ds(0, CHUNK * 16)],
                              semo[0]).wait()

    @pl.when(nc >= 2)
    def _d1():
        pltpu.make_async_copy(ob[1], out_ref.at[pl.ds(0, CHUNK * 16)],
                              semo[1]).wait()

    @pl.when(wid == (tail_owner + 1) % NW)
    def _chunk0():
        pltpu.sync_copy(tf_ref.at[:, pl.ds(0, CHUNK)], pb[0])
        _interleave(pb[0], ob[0], lane16, CHUNK // 16)
        pltpu.sync_copy(ob[0].at[pl.ds(16, CHUNK * 16 - 16)],
                        out_ref.at[pl.ds(0, CHUNK * 16 - 16)])

    @pl.when(wid == tail_owner)
    def _tail():
        pltpu.sync_copy(tail_ref, pb[1].at[:, pl.ds(0, rtail)])
        _interleave(pb[1], ob[1], lane16, (ntail + 15) // 16)
        pltpu.sync_copy(ob[1].at[pl.ds(0, ntail * 16)],
                        out_ref.at[pl.ds(nfull * (CHUNK * 16) - 16,
                                         ntail * 16)])


def _rel0_fn(tf_ref, tail_ref, out_ref, pb0, pb1, ob0, ob1, si0, si1, so0, so1):
    wid = lax.axis_index("subcore") * 2 + lax.axis_index("core")
    lane16 = lax.iota(jnp.int32, 16) * 16
    _rel_table(tf_ref, tail_ref, out_ref, (pb0, pb1), (ob0, ob1), (si0, si1),
               (so0, so1), lane16, wid, N0_FULL, N0_TAIL, 640, 7)


def _rels_fn(t1, t2, t3, t4, t5, t6, t7, t8, t9, t10, t11, t12,
             x1, x2, x3, x4, x5, x6, x7, x8, x9, x10, x11, x12,
             o1, o2, o3, o4, o5, o6, o7, o8, o9, o10, o11, o12,
             pb0, pb1, ob0, ob1, si0, si1, so0, so1):
    wid = lax.axis_index("subcore") * 2 + lax.axis_index("core")
    lane16 = lax.iota(jnp.int32, 16) * 16
    ins = (t1, t2, t3, t4, t5, t6, t7, t8, t9, t10, t11, t12)
    tails = (x1, x2, x3, x4, x5, x6, x7, x8, x9, x10, x11, x12)
    outs = (o1, o2, o3, o4, o5, o6, o7, o8, o9, o10, o11, o12)
    pb, ob = (pb0, pb1), (ob0, ob1)
    semi, semo = (si0, si1), (so0, so1)

    items = [(ins[i], outs[i], j) for i in range(len(ins)) for j in range(3)]

    def issue_in(n, p):
        tf_ref, _, j = items[n]
        c = j * NW + wid + 1
        pltpu.async_copy(tf_ref.at[:, pl.ds(c * CHUNK, CHUNK)], pb[p],
                         semi[p])

    issue_in(0, 0)
    for n, (tf_ref, out_ref, j) in enumerate(items):
        p = n % 2
        c = j * NW + wid + 1
        pltpu.make_async_copy(tf_ref.at[:, pl.ds(0, CHUNK)], pb[p],
                              semi[p]).wait()
        if n + 1 < len(items):
            issue_in(n + 1, 1 - p)
        if n >= 2:
            pltpu.make_async_copy(ob[p], out_ref.at[pl.ds(0, CHUNK * 16)],
                                  semo[p]).wait()
        _interleave(pb[p], ob[p], lane16, CHUNK // 16)
        pltpu.async_copy(ob[p], out_ref.at[pl.ds(c * (CHUNK * 16) - 16,
                                                 CHUNK * 16)], semo[p])

    for p in (0, 1):
        pltpu.make_async_copy(ob[p], outs[0].at[pl.ds(0, CHUNK * 16)],
                              semo[p]).wait()

    for i in range(len(ins)):
        @pl.when(wid == 3 + i)
        def _chunk0(tf_ref=ins[i], out_ref=outs[i]):
            pltpu.sync_copy(tf_ref.at[:, pl.ds(0, CHUNK)], pb[0])
            _interleave(pb[0], ob[0], lane16, CHUNK // 16)
            pltpu.sync_copy(ob[0].at[pl.ds(16, CHUNK * 16 - 16)],
                            out_ref.at[pl.ds(0, CHUNK * 16 - 16)])

        @pl.when(wid == 2 + i)
        def _tail(tail_ref=tails[i], out_ref=outs[i]):
            pltpu.sync_copy(tail_ref, pb[1].at[:, pl.ds(0, 768)])
            _interleave(pb[1], ob[1], lane16, (NS_TAIL + 15) // 16)
            pltpu.sync_copy(ob[1].at[pl.ds(0, NS_TAIL * 16)],
                            out_ref.at[pl.ds(NS_FULL * (CHUNK * 16) - 16,
                                             NS_TAIL * 16)])


def _hist_fn(uch_ref, t0_ref, out_ref, hidx0, hidx1, rows0, rows1, fh_v,
             semg0, semg1, semi0, semi1):
    wid = lax.axis_index("subcore") * 2 + lax.axis_index("core")
    base = wid * S_PER_W
    idx0 = wid * (S_PER_W * HIST)
    zero = jnp.zeros((16,), jnp.float32)
    hidx = (hidx0, hidx1)
    rows = (rows0, rows1)
    semg = (semg0, semg1)
    semi = (semi0, semi1)

    def idx_copy(g, p, sync):
        src = uch_ref.at[pl.ds(idx0 + g * IDX_PER_G, IDX_PER_G)]
        if sync:
            pltpu.sync_copy(src, hidx[p])
        else:
            pltpu.async_copy(src, hidx[p], semi[p])

    def fire(g, p):
        for j in range(IDX_PER_G // 128):
            pltpu.async_copy(t0_ref.at[hidx[p].at[pl.ds(j * 128, 128)]],
                             rows[p].at[pl.ds(j * 128, 128)], semg[p])

    def reduce(g, p):
        @pl.loop(0, G)
        def _sample(s):
            def body(j, accs):
                a0, a1 = accs
                o = s * HIST + j * 8
                for t in range(4):
                    a0 = a0 + rows[p][o + 2 * t]
                    a1 = a1 + rows[p][o + 2 * t + 1]
                return (a0, a1)

            a0, a1 = lax.fori_loop(0, HIST // 8, body, (zero, zero))
            fh_v[g * G + s] = a0 + a1

    idx_copy(0, 0, sync=True)
    fire(0, 0)
    idx_copy(1, 1, sync=False)

    @pl.loop(0, N_GROUPS // 2)
    def _g2(k):
        for p in range(2):
            g = k * 2 + p

            @pl.when(g + 1 < N_GROUPS)
            def _():
                pltpu.make_async_copy(
                    uch_ref.at[pl.ds(0, IDX_PER_G)], hidx[1 - p],
                    semi[1 - p]).wait()
                fire(g + 1, 1 - p)

            pltpu.make_async_copy(t0_ref.at[pl.ds(0, IDX_PER_G)], rows[p],
                                  semg[p]).wait()

            @pl.when(g + 2 < N_GROUPS)
            def _():
                idx_copy(g + 2, p, sync=False)

            reduce(g, p)

    pltpu.sync_copy(fh_v, out_ref.at[pl.ds(base, S_PER_W)])


def _sparse_fn(sp_ref, t0, t1, t2, t3, t4, t5, t6, t7, t8, t9, t10, t11, t12,
               out_ref, sidx0, sidx1, sidx2, srows0, srows1, srows2,
               scat0, scat1, semg0, semg1, semg2, sems0, sems1):
    tables = (t0, t1, t2, t3, t4, t5, t6, t7, t8, t9, t10, t11, t12)
    wid = lax.axis_index("subcore") * 2 + lax.axis_index("core")
    base = wid * S_PER_W
    lane = lax.iota(jnp.int32, 16)
    sidx = (sidx0, sidx1, sidx2)
    srows = (srows0, srows1, srows2)
    scat = (scat0, scat1)
    semg = (semg0, semg1, semg2)
    sems = (sems0, sems1)

    def fire_gathers(i):
        pg = i % 3
        pltpu.sync_copy(sp_ref.at[pl.ds(i * B + base, S_PER_W)], sidx[pg])
        for r in range(S_CHUNKS):
            pltpu.async_copy(tables[i].at[sidx[pg].at[pl.ds(r * 128, 128)]],
                             srows[pg].at[pl.ds(r * 128, 128)], semg[pg])

    def drain_scatters(i):
        pg, ps = i % 3, i % 2
        for r in range(S_CHUNKS):
            pltpu.make_async_copy(srows[pg].at[pl.ds(r * 128, 128)],
                                  out_ref.at[scat[ps].at[r]], sems[ps]).wait()

    fire_gathers(0)
    fire_gathers(1)
    for i in range(NUM_SPARSE):
        pg, ps = i % 3, i % 2
        pltpu.make_async_copy(tables[i].at[pl.ds(0, S_PER_W)], srows[pg],
                              semg[pg]).wait()
        if i >= 1:
            drain_scatters(i - 1)
        if i + 2 < NUM_SPARSE:
            fire_gathers(i + 2)

        @pl.loop(0, S_CHUNKS)
        def _fr(r):
            @pl.loop(0, 128, step=16)
            def _fc(c):
                k = base + r * 128 + c + lane
                scat[ps][r, pl.ds(c, 16)] = k * NUM_SPARSE + i

        for r in range(S_CHUNKS):
            pltpu.async_copy(srows[pg].at[pl.ds(r * 128, 128)],
                             out_ref.at[scat[ps].at[r]], sems[ps])

    drain_scatters(NUM_SPARSE - 1)


def _mlp_fn(fs_ref, h_ref, d_ref, w1s_ref, w1h_ref, w1d_ref, b1_ref, w2_ref,
            b2_ref, w3_ref, b3_ref, o_ref):
    h = jnp.dot(fs_ref[...], w1s_ref[...], preferred_element_type=jnp.float32)
    h = h + jnp.dot(h_ref[...], w1h_ref[...],
                    preferred_element_type=jnp.float32)
    h = h + jnp.dot(d_ref[...], w1d_ref[...],
                    preferred_element_type=jnp.float32)
    h = jnp.maximum(h + b1_ref[...], 0.0)
    h2 = jnp.dot(h, w2_ref[...], preferred_element_type=jnp.float32)
    h2 = jnp.maximum(h2 + b2_ref[...], 0.0)
    o_ref[...] = (jnp.dot(h2, w3_ref[...], preferred_element_type=jnp.float32)
                  + b3_ref[...])


_mesh = plsc.VectorSubcoreMesh(core_axis_name="core",
                               subcore_axis_name="subcore")

_REL_SCRATCH = [
    pltpu.VMEM((16, CHUNK), jnp.float32),
    pltpu.VMEM((16, CHUNK), jnp.float32),
    pltpu.VMEM((CHUNK * 16,), jnp.float32),
    pltpu.VMEM((CHUNK * 16,), jnp.float32),
    pltpu.SemaphoreType.DMA,
    pltpu.SemaphoreType.DMA,
    pltpu.SemaphoreType.DMA,
    pltpu.SemaphoreType.DMA,
]

_rel0 = pl.kernel(
    _rel0_fn,
    out_type=jax.ShapeDtypeStruct((N0_PAD * 16,), jnp.float32),
    mesh=_mesh,
    scratch_types=list(_REL_SCRATCH),
    compiler_params=_SC_PARAMS_TILED,
)

_rels = pl.kernel(
    _rels_fn,
    out_type=[jax.ShapeDtypeStruct((NS_PAD * 16,), jnp.float32)] * 12,
    mesh=_mesh,
    scratch_types=list(_REL_SCRATCH),
    compiler_params=_SC_PARAMS_TILED,
)

_hist = pl.kernel(
    _hist_fn,
    out_type=jax.ShapeDtypeStruct((B, EM), jnp.float32),
    mesh=_mesh,
    scratch_types=[
        pltpu.VMEM((IDX_PER_G,), jnp.int32),
        pltpu.VMEM((IDX_PER_G,), jnp.int32),
        pltpu.VMEM((IDX_PER_G, EM), jnp.float32),
        pltpu.VMEM((IDX_PER_G, EM), jnp.float32),
        pltpu.VMEM((S_PER_W, EM), jnp.float32),
        pltpu.SemaphoreType.DMA,
        pltpu.SemaphoreType.DMA,
        pltpu.SemaphoreType.DMA,
        pltpu.SemaphoreType.DMA,
    ],
    compiler_params=_SC_PARAMS,
)

_sparse = pl.kernel(
    _sparse_fn,
    out_type=jax.ShapeDtypeStruct((NUM_SPARSE * B, EM), jnp.float32),
    mesh=_mesh,
    scratch_types=[
        pltpu.VMEM((S_PER_W,), jnp.int32),
        pltpu.VMEM((S_PER_W,), jnp.int32),
        pltpu.VMEM((S_PER_W,), jnp.int32),
        pltpu.VMEM((S_PER_W, EM), jnp.float32),
        pltpu.VMEM((S_PER_W, EM), jnp.float32),
        pltpu.VMEM((S_PER_W, EM), jnp.float32),
        pltpu.VMEM((S_CHUNKS, 128), jnp.int32),
        pltpu.VMEM((S_CHUNKS, 128), jnp.int32),
        pltpu.SemaphoreType.DMA,
        pltpu.SemaphoreType.DMA,
        pltpu.SemaphoreType.DMA,
        pltpu.SemaphoreType.DMA,
        pltpu.SemaphoreType.DMA,
    ],
    compiler_params=_SC_PARAMS,
)


def kernel(sparse_features, dense_features, user_click_history, tables,
           fc1_w, fc1_b, fc2_w, fc2_b, fc3_w, fc3_b):
    uch1 = user_click_history.reshape(-1)
    sp1 = sparse_features.T.reshape(-1)

    t0t = tables[0].T
    tail0 = jnp.pad(t0t[:, N0_FULL * CHUNK:], ((0, 0), (0, 640 - N0_TAIL)))
    t0r = _rel0(t0t, tail0).reshape(N0_PAD, EM)
    stv = [t.T for t in tables[1:]]
    stails = [jnp.pad(t[:, NS_FULL * CHUNK:], ((0, 0), (0, 768 - NS_TAIL)))
              for t in stv]
    smalls = _rels(*stv, *stails)
    smalls = [s.reshape(NS_PAD, EM) for s in smalls]

    hist = _hist(uch1, t0r)
    featS = _sparse(sp1, t0r, *smalls).reshape(B, NUM_SPARSE * EM)

    w1s = fc1_w[:, :NUM_SPARSE * EM].T
    w1h = fc1_w[:, NUM_SPARSE * EM:(NUM_SPARSE + 1) * EM].T
    w1d = fc1_w[:, (NUM_SPARSE + 1) * EM:].T
    w2t = fc2_w.T
    w3t = fc3_w.T
    b1r = fc1_b.reshape(1, -1)
    b2r = fc2_b.reshape(1, -1)
    b3r = fc3_b.reshape(1, -1)

    BLK = 2048
    out = pl.pallas_call(
        _mlp_fn,
        grid=(B // BLK,),
        in_specs=[
            pl.BlockSpec((BLK, NUM_SPARSE * EM), lambda i: (i, 0)),
            pl.BlockSpec((BLK, EM), lambda i: (i, 0)),
            pl.BlockSpec((BLK, DENSE), lambda i: (i, 0)),
            pl.BlockSpec(w1s.shape, lambda i: (0, 0)),
            pl.BlockSpec(w1h.shape, lambda i: (0, 0)),
            pl.BlockSpec(w1d.shape, lambda i: (0, 0)),
            pl.BlockSpec(b1r.shape, lambda i: (0, 0)),
            pl.BlockSpec(w2t.shape, lambda i: (0, 0)),
            pl.BlockSpec(b2r.shape, lambda i: (0, 0)),
            pl.BlockSpec(w3t.shape, lambda i: (0, 0)),
            pl.BlockSpec(b3r.shape, lambda i: (0, 0)),
        ],
        out_specs=pl.BlockSpec((BLK, 2), lambda i: (i, 0)),
        out_shape=jax.ShapeDtypeStruct((B, 2), jnp.float32),
    )(featS, hist, dense_features, w1s, w1h, w1d, b1r, w2t, b2r, w3t, b3r)
    return out
